# baseline (speedup 1.0000x reference)
import functools

import jax
import jax.numpy as jnp
from jax import lax
from jax.experimental import pallas as pl
from jax.experimental.pallas import tpu as pltpu
from jax.experimental.pallas import tpu_sc as plsc


def _enc_argmin_body(x_ref, we_ref, be_ref, et_ref, z_ref, idx_ref,
                     z_s, m_s, a_s, *, bk):
    k = pl.program_id(1)
    nk = pl.num_programs(1)

    @pl.when(k == 0)
    def _():
        z = lax.dot_general(
            x_ref[...], we_ref[...],
            (((1,), (1,)), ((), ())),
            preferred_element_type=jnp.float32,
        ) + be_ref[...]
        z_s[...] = z
        z_ref[...] = z
        m_s[...] = jnp.full_like(m_s[...], -jnp.inf)
        a_s[...] = jnp.zeros_like(a_s[...])

    et = et_ref[...]
    rinv = lax.rsqrt(jnp.sum(et * et, axis=0, keepdims=True))
    s = lax.dot_general(
        z_s[...], et,
        (((1,), (0,)), ((), ())),
        preferred_element_type=jnp.float32,
    )
    m = s * rinv
    tm = jnp.max(m, axis=1, keepdims=True)
    iota = lax.broadcasted_iota(jnp.int32, m.shape, 1)
    ta = jnp.min(jnp.where(m == tm, iota, jnp.int32(2 ** 30)),
                 axis=1, keepdims=True) + k * bk
    upd = tm > m_s[...]
    a_s[...] = jnp.where(upd, ta, a_s[...])
    m_s[...] = jnp.where(upd, tm, m_s[...])

    @pl.when(k == nk - 1)
    def _():
        idx_ref[...] = a_s[...]


def _enc_argmin(x, w_enc, b_enc, emb_t, *, bb, bkk):
    b, d = x.shape
    l, kk = emb_t.shape
    nb, nk = b // bb, kk // bkk
    z, idx = pl.pallas_call(
        functools.partial(_enc_argmin_body, bk=bkk),
        grid=(nb, nk),
        in_specs=[
            pl.BlockSpec((bb, d), lambda i, j: (i, 0)),
            pl.BlockSpec((l, d), lambda i, j: (0, 0)),
            pl.BlockSpec((1, l), lambda i, j: (0, 0)),
            pl.BlockSpec((l, bkk), lambda i, j: (0, j)),
        ],
        out_specs=[
            pl.BlockSpec((bb, l), lambda i, j: (i, 0)),
            pl.BlockSpec((bb, 1), lambda i, j: (i, 0)),
        ],
        out_shape=[
            jax.ShapeDtypeStruct((b, l), jnp.float32),
            jax.ShapeDtypeStruct((b, 1), jnp.int32),
        ],
        scratch_shapes=[
            pltpu.VMEM((bb, l), jnp.float32),
            pltpu.VMEM((bb, 1), jnp.float32),
            pltpu.VMEM((bb, 1), jnp.int32),
        ],
        compiler_params=pltpu.CompilerParams(
            dimension_semantics=("parallel", "arbitrary"),
        ),
    )(x, w_enc, b_enc.reshape(1, l), emb_t)
    return z, idx.reshape(b)


def _dec_table_body(emb_ref, wd_ref, bd_ref, out_ref):
    out_ref[...] = lax.dot_general(
        emb_ref[...], wd_ref[...],
        (((1,), (1,)), ((), ())),
        preferred_element_type=jnp.float32,
    ) + bd_ref[...]


def _dec_table(emb, w_dec, b_dec, *, bkk):
    kk, l = emb.shape
    d = w_dec.shape[0]
    return pl.pallas_call(
        _dec_table_body,
        grid=(kk // bkk,),
        in_specs=[
            pl.BlockSpec((bkk, l), lambda i: (i, 0)),
            pl.BlockSpec((d, l), lambda i: (0, 0)),
            pl.BlockSpec((1, d), lambda i: (0, 0)),
        ],
        out_specs=pl.BlockSpec((bkk, d), lambda i: (i, 0)),
        out_shape=jax.ShapeDtypeStruct((kk, d), jnp.float32),
        compiler_params=pltpu.CompilerParams(
            dimension_semantics=("parallel",),
        ),
    )(emb, w_dec, b_dec.reshape(1, d))


_NC, _NS = 2, 16
_NW = _NC * _NS
_CHUNK = 32


def _gather_call(idx, emb, emb_dec):
    b = idx.shape[0]
    kk, l = emb.shape
    d = emb_dec.shape[1]
    b_per_w = b // _NW
    n_chunks = b_per_w // _CHUNK
    mesh = plsc.VectorSubcoreMesh(core_axis_name="c", subcore_axis_name="s")

    @functools.partial(
        pl.kernel,
        mesh=mesh,
        out_type=[
            jax.ShapeDtypeStruct((b, l), jnp.float32),
            jax.ShapeDtypeStruct((b, d), jnp.float32),
        ],
        scratch_types=[
            pltpu.VMEM((b_per_w,), jnp.int32),
            pltpu.VMEM((_CHUNK, l), jnp.float32),
            pltpu.VMEM((_CHUNK, l), jnp.float32),
            pltpu.VMEM((_CHUNK, d), jnp.float32),
            pltpu.VMEM((_CHUNK, d), jnp.float32),
            pltpu.SemaphoreType.DMA,
            pltpu.SemaphoreType.DMA,
            pltpu.SemaphoreType.DMA,
            pltpu.SemaphoreType.DMA,
        ],
    )
    def gather_k(idx_hbm, emb_hbm, dec_hbm, zq_hbm, xr_hbm,
                 idx_v, zbuf0, zbuf1, xbuf0, xbuf1,
                 sem_z0, sem_z1, sem_x0, sem_x1):
        wid = lax.axis_index("s") * _NC + lax.axis_index("c")
        base = wid * b_per_w
        pltpu.sync_copy(idx_hbm.at[pl.ds(base, b_per_w)], idx_v)
        bufs = [(zbuf0, xbuf0, sem_z0, sem_x0),
                (zbuf1, xbuf1, sem_z1, sem_x1)]

        def start(c):
            zb, xb, sz, sx = bufs[c % 2]
            idx_c = idx_v.at[pl.ds(c * _CHUNK, _CHUNK)]
            return (pltpu.async_copy(emb_hbm.at[idx_c], zb, sz),
                    pltpu.async_copy(dec_hbm.at[idx_c], xb, sx))

        inflight = {0: start(0)}
        for c in range(n_chunks):
            if c + 1 < n_chunks:
                inflight[c + 1] = start(c + 1)
            cp_z, cp_x = inflight.pop(c)
            zb, xb, _, _ = bufs[c % 2]
            off = c * _CHUNK
            cp_z.wait()
            pltpu.sync_copy(zb, zq_hbm.at[pl.ds(base + off, _CHUNK)])
            cp_x.wait()
            pltpu.sync_copy(xb, xr_hbm.at[pl.ds(base + off, _CHUNK)])

    return gather_k(idx, emb, emb_dec)


def kernel(x, W_enc, b_enc, emb, W_dec, b_dec):
    b, d = x.shape
    kk, l = emb.shape
    emb_t = emb.T
    emb_dec = _dec_table(emb, W_dec, b_dec, bkk=min(1024, kk))
    if b % (2 * 32 * _CHUNK) == 0:
        h = b // 2
        z0, idx0 = _enc_argmin(x[:h], W_enc, b_enc, emb_t,
                               bb=min(1024, h), bkk=min(4096, kk))
        zq0, xr0 = _gather_call(idx0, emb, emb_dec)
        z1, idx1 = _enc_argmin(x[h:], W_enc, b_enc, emb_t,
                               bb=min(1024, h), bkk=min(4096, kk))
        zq1, xr1 = _gather_call(idx1, emb, emb_dec)
        z = jnp.concatenate([z0, z1], axis=0)
        idx = jnp.concatenate([idx0, idx1], axis=0)
        z_q = jnp.concatenate([zq0, zq1], axis=0)
        x_recon = jnp.concatenate([xr0, xr1], axis=0)
    else:
        z, idx = _enc_argmin(x, W_enc, b_enc, emb_t,
                             bb=min(1024, b), bkk=min(4096, kk))
        z_q, x_recon = _gather_call(idx, emb, emb_dec)
    return (x_recon, z, z_q, idx)


# speedup vs baseline: 1.2558x; 1.2558x over previous
import functools

import jax
import jax.numpy as jnp
from jax import lax
from jax.experimental import pallas as pl
from jax.experimental.pallas import tpu as pltpu
from jax.experimental.pallas import tpu_sc as plsc


def _enc_argmin_body(x_ref, we_ref, be_ref, et_ref, z_ref, idx_ref,
                     z_s, m_s, a_s, *, bk):
    k = pl.program_id(1)
    nk = pl.num_programs(1)

    @pl.when(k == 0)
    def _():
        z = lax.dot_general(
            x_ref[...], we_ref[...],
            (((1,), (1,)), ((), ())),
            preferred_element_type=jnp.float32,
        ) + be_ref[...]
        z_s[...] = z
        z_ref[...] = z
        m_s[...] = jnp.full_like(m_s[...], -jnp.inf)
        a_s[...] = jnp.zeros_like(a_s[...])

    et = et_ref[...]
    rinv = lax.rsqrt(jnp.sum(et * et, axis=0, keepdims=True))
    s = lax.dot_general(
        z_s[...], et,
        (((1,), (0,)), ((), ())),
        preferred_element_type=jnp.float32,
    )
    m = s * rinv
    tm = jnp.max(m, axis=1, keepdims=True)
    iota = lax.broadcasted_iota(jnp.int32, m.shape, 1)
    ta = jnp.min(jnp.where(m == tm, iota, jnp.int32(2 ** 30)),
                 axis=1, keepdims=True) + k * bk
    upd = tm > m_s[...]
    a_s[...] = jnp.where(upd, ta, a_s[...])
    m_s[...] = jnp.where(upd, tm, m_s[...])

    @pl.when(k == nk - 1)
    def _():
        idx_ref[...] = a_s[...]


def _enc_argmin(x, w_enc, b_enc, emb_t, *, bb, bkk):
    b, d = x.shape
    l, kk = emb_t.shape
    nb, nk = b // bb, kk // bkk
    z, idx = pl.pallas_call(
        functools.partial(_enc_argmin_body, bk=bkk),
        grid=(nb, nk),
        in_specs=[
            pl.BlockSpec((bb, d), lambda i, j: (i, 0)),
            pl.BlockSpec((l, d), lambda i, j: (0, 0)),
            pl.BlockSpec((1, l), lambda i, j: (0, 0)),
            pl.BlockSpec((l, bkk), lambda i, j: (0, j)),
        ],
        out_specs=[
            pl.BlockSpec((bb, l), lambda i, j: (i, 0)),
            pl.BlockSpec((bb, 1), lambda i, j: (i, 0)),
        ],
        out_shape=[
            jax.ShapeDtypeStruct((b, l), jnp.float32),
            jax.ShapeDtypeStruct((b, 1), jnp.int32),
        ],
        scratch_shapes=[
            pltpu.VMEM((bb, l), jnp.float32),
            pltpu.VMEM((bb, 1), jnp.float32),
            pltpu.VMEM((bb, 1), jnp.int32),
        ],
        compiler_params=pltpu.CompilerParams(
            dimension_semantics=("parallel", "arbitrary"),
        ),
    )(x, w_enc, b_enc.reshape(1, l), emb_t)
    return z, idx.reshape(b)


def _dec_table_body(emb_ref, wd_ref, bd_ref, out_ref):
    out_ref[...] = lax.dot_general(
        emb_ref[...], wd_ref[...],
        (((1,), (1,)), ((), ())),
        preferred_element_type=jnp.float32,
    ) + bd_ref[...]


def _dec_table(emb, w_dec, b_dec, *, bkk):
    kk, l = emb.shape
    d = w_dec.shape[0]
    return pl.pallas_call(
        _dec_table_body,
        grid=(kk // bkk,),
        in_specs=[
            pl.BlockSpec((bkk, l), lambda i: (i, 0)),
            pl.BlockSpec((d, l), lambda i: (0, 0)),
            pl.BlockSpec((1, d), lambda i: (0, 0)),
        ],
        out_specs=pl.BlockSpec((bkk, d), lambda i: (i, 0)),
        out_shape=jax.ShapeDtypeStruct((kk, d), jnp.float32),
        compiler_params=pltpu.CompilerParams(
            dimension_semantics=("parallel",),
        ),
    )(emb, w_dec, b_dec.reshape(1, d))


_NC, _NS = 2, 16
_NW = _NC * _NS
_CHUNK = 32


def _gather_call(idx, emb, emb_dec):
    b = idx.shape[0]
    kk, l = emb.shape
    d = emb_dec.shape[1]
    b_per_w = b // _NW
    n_chunks = b_per_w // _CHUNK
    mesh = plsc.VectorSubcoreMesh(core_axis_name="c", subcore_axis_name="s")

    @functools.partial(
        pl.kernel,
        mesh=mesh,
        out_type=[
            jax.ShapeDtypeStruct((b, l), jnp.float32),
            jax.ShapeDtypeStruct((b, d), jnp.float32),
        ],
        scratch_types=[
            pltpu.VMEM((b_per_w,), jnp.int32),
            pltpu.VMEM((_CHUNK, l), jnp.float32),
            pltpu.VMEM((_CHUNK, l), jnp.float32),
            pltpu.VMEM((_CHUNK, d), jnp.float32),
            pltpu.VMEM((_CHUNK, d), jnp.float32),
            pltpu.SemaphoreType.DMA,
            pltpu.SemaphoreType.DMA,
            pltpu.SemaphoreType.DMA,
            pltpu.SemaphoreType.DMA,
        ],
    )
    def gather_k(idx_hbm, emb_hbm, dec_hbm, zq_hbm, xr_hbm,
                 idx_v, zbuf0, zbuf1, xbuf0, xbuf1,
                 sem_z0, sem_z1, sem_x0, sem_x1):
        wid = lax.axis_index("s") * _NC + lax.axis_index("c")
        base = wid * b_per_w
        pltpu.sync_copy(idx_hbm.at[pl.ds(base, b_per_w)], idx_v)
        bufs = [(zbuf0, xbuf0, sem_z0, sem_x0),
                (zbuf1, xbuf1, sem_z1, sem_x1)]

        def start(c):
            zb, xb, sz, sx = bufs[c % 2]
            idx_c = idx_v.at[pl.ds(c * _CHUNK, _CHUNK)]
            return (pltpu.async_copy(emb_hbm.at[idx_c], zb, sz),
                    pltpu.async_copy(dec_hbm.at[idx_c], xb, sx))

        inflight = {0: start(0)}
        for c in range(n_chunks):
            if c + 1 < n_chunks:
                inflight[c + 1] = start(c + 1)
            cp_z, cp_x = inflight.pop(c)
            zb, xb, _, _ = bufs[c % 2]
            off = c * _CHUNK
            cp_z.wait()
            pltpu.sync_copy(zb, zq_hbm.at[pl.ds(base + off, _CHUNK)])
            cp_x.wait()
            pltpu.sync_copy(xb, xr_hbm.at[pl.ds(base + off, _CHUNK)])

    return gather_k(idx, emb, emb_dec)


def kernel(x, W_enc, b_enc, emb, W_dec, b_dec):
    b, d = x.shape
    kk, l = emb.shape
    z, idx = _enc_argmin(x, W_enc, b_enc, emb.T,
                         bb=min(1024, b), bkk=min(4096, kk))
    emb_dec = _dec_table(emb, W_dec, b_dec, bkk=min(1024, kk))
    z_q, x_recon = _gather_call(idx, emb, emb_dec)
    return (x_recon, z, z_q, idx)
